# baseline (speedup 1.0000x reference)
import numpy as np
import jax
import jax.numpy as jnp
from jax import lax
from jax.experimental import pallas as pl
from jax.experimental.pallas import tpu as pltpu
from jax.experimental.pallas import tpu_sc as plsc

DHW = 128
KCONST = (np.pi / 3.5) ** 2
N_ATOMS = 2000
NC, NS, L = 2, 16, 16
NW = NC * NS
NZ = 2
NPASS = DHW // (NW * NZ)
PLANE = DHW * DHW
SLAB = NZ * PLANE
QSTRIDE = 6144

_offs = sorted(
    [(oy, ox) for oy in range(-2, 3) for ox in range(-2, 3) if oy * oy + ox * ox <= 6],
    key=lambda p: p[0] * p[0] + p[1] * p[1],
)
_oy = np.array([o[0] for o in _offs] + [0] * 11, np.int32)
_ox = np.array([o[1] for o in _offs] + [0] * 11, np.int32)
_r2 = np.array([o[0] ** 2 + o[1] ** 2 for o in _offs] + [999] * 11, np.int64)

_ITAB = np.concatenate([
    (_oy[0:16] * DHW + _ox[0:16]).astype(np.int32),
    (_oy[16:32] * DHW + _ox[16:32]).astype(np.int32),
    _oy[0:16], _oy[16:32], _ox[0:16], _ox[16:32],
    (np.arange(16) < 5).astype(np.int32),
    (_r2[0:16] <= 2).astype(np.int32),
]).astype(np.int32)

_w = lambda adz, g: np.exp(
    -KCONST * (adz * adz + _r2[g * 16:(g + 1) * 16].astype(np.float64))
).astype(np.float32)
_FTAB = np.concatenate(
    [_w(0, 0), _w(0, 1), _w(1, 0), _w(1, 1), _w(2, 0), np.zeros(16, np.float32)])


def _sc_body(src_hbm, tgt_hbm, itab_hbm, ftab_hbm, out_hbm,
             sg, tg, cbuf, cells, wlz, wly, wlx, itab, ftab, outv):
    cid = lax.axis_index("c")
    sid = lax.axis_index("s")
    wid = sid * NC + cid

    pltpu.sync_copy(itab_hbm, itab)
    pltpu.sync_copy(ftab_hbm, ftab)

    off_v = [itab[pl.ds(0, L)], itab[pl.ds(L, L)]]
    oy_v = [itab[pl.ds(2 * L, L)], itab[pl.ds(3 * L, L)]]
    ox_v = [itab[pl.ds(4 * L, L)], itab[pl.ds(5 * L, L)]]
    g1act = itab[pl.ds(6 * L, L)] != 0
    g0act9 = itab[pl.ds(7 * L, L)] != 0
    wgt_v = [[ftab[pl.ds(0, L)], ftab[pl.ds(L, L)]],
             [ftab[pl.ds(2 * L, L)], ftab[pl.ds(3 * L, L)]],
             [ftab[pl.ds(4 * L, L)], None]]
    act_v = [[None, g1act], [None, g1act], [g0act9, None]]
    zero_v = ftab[pl.ds(5 * L, L)]

    for q, hbm in enumerate((src_hbm, tgt_hbm)):
        pltpu.sync_copy(hbm, cbuf)
        row = q * QSTRIDE

        def conv_body(i, _, row=row):
            v = cbuf[pl.ds(i * L, L)] * np.float32(DHW)
            cells[pl.ds(row + i * L, L)] = v.astype(jnp.int32)
            return 0

        lax.fori_loop(0, 3 * N_ATOMS // L, conv_body, 0)

    def splat_atoms(grid, cbase, s0):
        def scan_body(i, cnt):
            zv = cells[pl.ds(cbase + i * L, L)]
            hitv = (zv >= s0 - 2) & (zv <= s0 + NZ + 1)
            c = plsc.all_reduce_population_count(hitv)[0]

            @pl.when(c > 0)
            def _():
                yv = cells[pl.ds(cbase + N_ATOMS + i * L, L)]
                xv = cells[pl.ds(cbase + 2 * N_ATOMS + i * L, L)]
                plsc.store_compressed(wlz.at[pl.ds(cnt, L)], zv, mask=hitv)
                plsc.store_compressed(wly.at[pl.ds(cnt, L)], yv, mask=hitv)
                plsc.store_compressed(wlx.at[pl.ds(cnt, L)], xv, mask=hitv)

            return cnt + c

        nhit = lax.fori_loop(0, N_ATOMS // L, scan_body, 0)

        def body(a, _):
            zc = wlz[pl.ds(a, L)][0]
            yc = wly[pl.ds(a, L)][0]
            xc = wlx[pl.ds(a, L)][0]
            byx = yc * DHW + xc
            myx = []
            for g in range(2):
                y = yc + oy_v[g]
                x = xc + ox_v[g]
                myx.append((y >= 0) & (y < DHW) & (x >= 0) & (x < DHW))
            for dz in (-2, -1, 0, 1, 2):
                adz = abs(dz)
                lz = zc + dz - s0

                @pl.when((lz >= 0) & (lz < NZ))
                def _(lz=lz, adz=adz):
                    base = lz * PLANE + byx
                    ngroups = 2 if adz <= 1 else 1
                    for g in range(ngroups):
                        idx = base + off_v[g]
                        m = myx[g]
                        if act_v[adz][g] is not None:
                            m = m & act_v[adz][g]
                        cur = plsc.load_gather(grid, [idx], mask=m)
                        plsc.store_scatter(
                            grid, [idx], jnp.maximum(cur, wgt_v[adz][g]), mask=m)

            return 0

        lax.fori_loop(0, nhit, body, 0)

    def zbody(i, _):
        sg[pl.ds(i * L, L)] = zero_v
        tg[pl.ds(i * L, L)] = zero_v
        return 0

    def pass_body(p, accs):
        acc_ss, acc_st = accs
        s0 = p * (NW * NZ) + wid * NZ

        lax.fori_loop(0, SLAB // L, zbody, 0)

        splat_atoms(sg, 0, s0)
        splat_atoms(tg, QSTRIDE, s0)

        def rbody(i, carry):
            css, cst = carry
            s = sg[pl.ds(i * L, L)]
            t = tg[pl.ds(i * L, L)]
            return (css + s * s, cst + s * t)

        return lax.fori_loop(0, SLAB // L, rbody, (acc_ss, acc_st))

    acc_ss, acc_st = lax.fori_loop(0, NPASS, pass_body, (zero_v, zero_v))

    for i in range(2 * DHW // L):
        outv[pl.ds(i * L, L)] = zero_v
    outv[pl.ds(0, L)] = acc_ss
    outv[pl.ds(DHW, L)] = acc_st
    pltpu.sync_copy(outv.at[pl.ds(0, DHW)], out_hbm.at[pl.ds(wid * DHW, DHW)])
    pltpu.sync_copy(outv.at[pl.ds(DHW, DHW)],
                    out_hbm.at[pl.ds((NW + wid) * DHW, DHW)])


@jax.jit
def _run(srcc, tgtt):
    mesh = plsc.VectorSubcoreMesh(
        core_axis_name="c", subcore_axis_name="s", num_cores=NC, num_subcores=NS)
    out = pl.kernel(
        _sc_body,
        out_type=jax.ShapeDtypeStruct((2 * NW * DHW,), jnp.float32),
        mesh=mesh,
        compiler_params=pltpu.CompilerParams(needs_layout_passes=False),
        scratch_types=[
            pltpu.VMEM((SLAB,), jnp.float32),
            pltpu.VMEM((SLAB,), jnp.float32),
            pltpu.VMEM((3 * N_ATOMS,), jnp.float32),
            pltpu.VMEM((2 * QSTRIDE,), jnp.int32),
            pltpu.VMEM((N_ATOMS + L,), jnp.int32),
            pltpu.VMEM((N_ATOMS + L,), jnp.int32),
            pltpu.VMEM((N_ATOMS + L,), jnp.int32),
            pltpu.VMEM((8 * L,), jnp.int32),
            pltpu.VMEM((6 * L,), jnp.float32),
            pltpu.VMEM((2 * DHW,), jnp.float32),
        ],
    )(srcc, tgtt, jnp.asarray(_ITAB), jnp.asarray(_FTAB))
    halves = out.reshape(2, NW * DHW)
    return jnp.sum(halves[0]) - jnp.sum(halves[1])


def kernel(src, tgt):
    return _run(src.reshape(3 * N_ATOMS), tgt.reshape(3 * N_ATOMS))


# speedup vs baseline: 1.3374x; 1.3374x over previous
import numpy as np
import jax
import jax.numpy as jnp
from jax import lax
from jax.experimental import pallas as pl
from jax.experimental.pallas import tpu as pltpu
from jax.experimental.pallas import tpu_sc as plsc

DHW = 128
KCONST = (np.pi / 3.5) ** 2
N_ATOMS = 2000
NC, NS, L = 2, 16, 16
NW = NC * NS
NZ = 2
NPASS = DHW // (NW * NZ)
PLANE = DHW * DHW
SLAB = NZ * PLANE
QSTRIDE = 6144

_offs = sorted(
    [(oy, ox) for oy in range(-2, 3) for ox in range(-2, 3) if oy * oy + ox * ox <= 6],
    key=lambda p: p[0] * p[0] + p[1] * p[1],
)
_oy = np.array([o[0] for o in _offs] + [0] * 11, np.int32)
_ox = np.array([o[1] for o in _offs] + [0] * 11, np.int32)
_r2 = np.array([o[0] ** 2 + o[1] ** 2 for o in _offs] + [999] * 11, np.int64)

_ITAB = np.concatenate([
    (_oy[0:16] * DHW + _ox[0:16]).astype(np.int32),
    (_oy[16:32] * DHW + _ox[16:32]).astype(np.int32),
    _oy[0:16], _oy[16:32], _ox[0:16], _ox[16:32],
    (np.arange(16) < 5).astype(np.int32),
    (_r2[0:16] <= 2).astype(np.int32),
]).astype(np.int32)

_w = lambda adz, g: np.exp(
    -KCONST * (adz * adz + _r2[g * 16:(g + 1) * 16].astype(np.float64))
).astype(np.float32)
_FTAB = np.concatenate(
    [_w(0, 0), _w(0, 1), _w(1, 0), _w(1, 1), _w(2, 0), np.zeros(16, np.float32)])


def _sc_body(src_hbm, tgt_hbm, itab_hbm, ftab_hbm, out_hbm,
             sg, tg, cbuf, cells, wlz, wly, wlx, itab, ftab, outv):
    cid = lax.axis_index("c")
    sid = lax.axis_index("s")
    wid = sid * NC + cid

    pltpu.sync_copy(itab_hbm, itab)
    pltpu.sync_copy(ftab_hbm, ftab)

    off_v = [itab[pl.ds(0, L)], itab[pl.ds(L, L)]]
    oy_v = [itab[pl.ds(2 * L, L)], itab[pl.ds(3 * L, L)]]
    ox_v = [itab[pl.ds(4 * L, L)], itab[pl.ds(5 * L, L)]]
    g1act = itab[pl.ds(6 * L, L)] != 0
    g0act9 = itab[pl.ds(7 * L, L)] != 0
    wgt_v = [[ftab[pl.ds(0, L)], ftab[pl.ds(L, L)]],
             [ftab[pl.ds(2 * L, L)], ftab[pl.ds(3 * L, L)]],
             [ftab[pl.ds(4 * L, L)], None]]
    act_v = [[None, g1act], [None, g1act], [g0act9, None]]
    zero_v = ftab[pl.ds(5 * L, L)]

    for q, hbm in enumerate((src_hbm, tgt_hbm)):
        pltpu.sync_copy(hbm, cbuf)
        row = q * QSTRIDE

        def conv_body(i, _, row=row):
            v = cbuf[pl.ds(i * L, L)] * np.float32(DHW)
            cells[pl.ds(row + i * L, L)] = v.astype(jnp.int32)
            return 0

        lax.fori_loop(0, 3 * N_ATOMS // L, conv_body, 0)

    def splat_atoms(grid, cbase, s0):
        def scan_body(i, cnt):
            zv = cells[pl.ds(cbase + i * L, L)]
            hitv = (zv >= s0 - 2) & (zv <= s0 + NZ + 1)
            c = plsc.all_reduce_population_count(hitv)[0]

            @pl.when(c > 0)
            def _():
                yv = cells[pl.ds(cbase + N_ATOMS + i * L, L)]
                xv = cells[pl.ds(cbase + 2 * N_ATOMS + i * L, L)]
                plsc.store_compressed(wlz.at[pl.ds(cnt, L)], zv, mask=hitv)
                plsc.store_compressed(wly.at[pl.ds(cnt, L)], yv, mask=hitv)
                plsc.store_compressed(wlx.at[pl.ds(cnt, L)], xv, mask=hitv)

            return cnt + c

        nhit = lax.fori_loop(0, N_ATOMS // L, scan_body, 0)

        def body(a, _):
            zc = wlz[pl.ds(a, L)][0]
            yc = wly[pl.ds(a, L)][0]
            xc = wlx[pl.ds(a, L)][0]
            byx = yc * DHW + xc
            myx = []
            for g in range(2):
                y = yc + oy_v[g]
                x = xc + ox_v[g]
                myx.append((y >= 0) & (y < DHW) & (x >= 0) & (x < DHW))
            for dz in (-2, -1, 0, 1, 2):
                adz = abs(dz)
                lz = zc + dz - s0

                @pl.when((lz >= 0) & (lz < NZ))
                def _(lz=lz, adz=adz):
                    base = lz * PLANE + byx
                    ngroups = 2 if adz <= 1 else 1
                    for g in range(ngroups):
                        idx = base + off_v[g]
                        m = myx[g]
                        if act_v[adz][g] is not None:
                            m = m & act_v[adz][g]
                        cur = plsc.load_gather(grid, [idx], mask=m)
                        plsc.store_scatter(
                            grid, [idx], jnp.maximum(cur, wgt_v[adz][g]), mask=m)

            return 0

        lax.fori_loop(0, nhit, body, 0)

    def zbody(i, _):
        sg[pl.ds(i * L, L)] = zero_v
        tg[pl.ds(i * L, L)] = zero_v
        return 0

    def pass_body(p, accs):
        acc_ss, acc_st = accs
        s0 = p * (NW * NZ) + wid * NZ

        lax.fori_loop(0, SLAB // L, zbody, 0, unroll=8)

        splat_atoms(sg, 0, s0)
        splat_atoms(tg, QSTRIDE, s0)

        def rbody(i, carry):
            css, cst = carry
            s = sg[pl.ds(i * L, L)]
            t = tg[pl.ds(i * L, L)]
            return (css + s * s, cst + s * t)

        return lax.fori_loop(0, SLAB // L, rbody, (acc_ss, acc_st), unroll=8)

    acc_ss, acc_st = lax.fori_loop(0, NPASS, pass_body, (zero_v, zero_v))

    for i in range(2 * DHW // L):
        outv[pl.ds(i * L, L)] = zero_v
    outv[pl.ds(0, L)] = acc_ss
    outv[pl.ds(DHW, L)] = acc_st
    pltpu.sync_copy(outv.at[pl.ds(0, DHW)], out_hbm.at[pl.ds(wid * DHW, DHW)])
    pltpu.sync_copy(outv.at[pl.ds(DHW, DHW)],
                    out_hbm.at[pl.ds((NW + wid) * DHW, DHW)])


@jax.jit
def _run(srcc, tgtt):
    mesh = plsc.VectorSubcoreMesh(
        core_axis_name="c", subcore_axis_name="s", num_cores=NC, num_subcores=NS)
    out = pl.kernel(
        _sc_body,
        out_type=jax.ShapeDtypeStruct((2 * NW * DHW,), jnp.float32),
        mesh=mesh,
        compiler_params=pltpu.CompilerParams(needs_layout_passes=False),
        scratch_types=[
            pltpu.VMEM((SLAB,), jnp.float32),
            pltpu.VMEM((SLAB,), jnp.float32),
            pltpu.VMEM((3 * N_ATOMS,), jnp.float32),
            pltpu.VMEM((2 * QSTRIDE,), jnp.int32),
            pltpu.VMEM((N_ATOMS + L,), jnp.int32),
            pltpu.VMEM((N_ATOMS + L,), jnp.int32),
            pltpu.VMEM((N_ATOMS + L,), jnp.int32),
            pltpu.VMEM((8 * L,), jnp.int32),
            pltpu.VMEM((6 * L,), jnp.float32),
            pltpu.VMEM((2 * DHW,), jnp.float32),
        ],
    )(srcc, tgtt, jnp.asarray(_ITAB), jnp.asarray(_FTAB))
    halves = out.reshape(2, NW * DHW)
    return jnp.sum(halves[0]) - jnp.sum(halves[1])


def kernel(src, tgt):
    return _run(src.reshape(3 * N_ATOMS), tgt.reshape(3 * N_ATOMS))


# speedup vs baseline: 1.6131x; 1.2062x over previous
import numpy as np
import jax
import jax.numpy as jnp
from jax import lax
from jax.experimental import pallas as pl
from jax.experimental.pallas import tpu as pltpu
from jax.experimental.pallas import tpu_sc as plsc

DHW = 128
KCONST = (np.pi / 3.5) ** 2
N_ATOMS = 2000
NC, NS, L = 2, 16, 16
NW = NC * NS
NZ = 2
NPASS = DHW // (NW * NZ)
PLANE = DHW * DHW
SLAB = NZ * PLANE
QSTRIDE = 6144
WLCAP = N_ATOMS + L

_offs = sorted(
    [(oy, ox) for oy in range(-2, 3) for ox in range(-2, 3) if oy * oy + ox * ox <= 6],
    key=lambda p: p[0] * p[0] + p[1] * p[1],
)
_oy = np.array([o[0] for o in _offs] + [0] * 11, np.int32)
_ox = np.array([o[1] for o in _offs] + [0] * 11, np.int32)
_r2 = np.array([o[0] ** 2 + o[1] ** 2 for o in _offs] + [999] * 11, np.int64)

_ITAB = np.concatenate([
    (_oy[0:16] * DHW + _ox[0:16]).astype(np.int32),
    (_oy[16:32] * DHW + _ox[16:32]).astype(np.int32),
    _oy[0:16], _oy[16:32], _ox[0:16], _ox[16:32],
]).astype(np.int32)

_wrow = lambda adz, g: np.where(
    adz * adz + _r2[g * 16:(g + 1) * 16] <= 6,
    np.exp(-KCONST * (adz * adz + _r2[g * 16:(g + 1) * 16].astype(np.float64))),
    0.0).astype(np.float32)
_FTAB = np.concatenate(
    [_wrow(a, g) for a in range(3) for g in range(2)] + [np.zeros(16, np.float32)])


def _sc_body(src_hbm, tgt_hbm, itab_hbm, ftab_hbm, out_hbm,
             sg, tg, cbuf, cells, wlz, wly, wlx, itab, ftab, outv):
    cid = lax.axis_index("c")
    sid = lax.axis_index("s")
    wid = sid * NC + cid

    pltpu.sync_copy(itab_hbm, itab)
    pltpu.sync_copy(ftab_hbm, ftab)

    off_v = [itab[pl.ds(0, L)], itab[pl.ds(L, L)]]
    oy_v = [itab[pl.ds(2 * L, L)], itab[pl.ds(3 * L, L)]]
    ox_v = [itab[pl.ds(4 * L, L)], itab[pl.ds(5 * L, L)]]
    zero_v = ftab[pl.ds(6 * L, L)]

    s0s = [p * (NW * NZ) + wid * NZ for p in range(NPASS)]

    for q, hbm in enumerate((src_hbm, tgt_hbm)):
        pltpu.sync_copy(hbm, cbuf)
        row = q * QSTRIDE

        def conv_body(i, _, row=row):
            v = cbuf[pl.ds(i * L, L)] * np.float32(DHW)
            cells[pl.ds(row + i * L, L)] = v.astype(jnp.int32)
            return 0

        lax.fori_loop(0, 3 * N_ATOMS // L, conv_body, 0, unroll=4)

    def scan_cloud(cbase, wbase):
        def sb(i, cnts):
            zv = cells[pl.ds(cbase + i * L, L)]
            yv = cells[pl.ds(cbase + N_ATOMS + i * L, L)]
            xv = cells[pl.ds(cbase + 2 * N_ATOMS + i * L, L)]
            new = []
            for p in range(NPASS):
                h = (zv >= s0s[p] - 2) & (zv <= s0s[p] + NZ + 1)
                c = cnts[p]
                at = wbase + p * WLCAP + c
                plsc.store_compressed(wlz.at[pl.ds(at, L)], zv, mask=h)
                plsc.store_compressed(wly.at[pl.ds(at, L)], yv, mask=h)
                plsc.store_compressed(wlx.at[pl.ds(at, L)], xv, mask=h)
                new.append(c + plsc.all_reduce_population_count(h)[0])
            return tuple(new)

        return lax.fori_loop(0, N_ATOMS // L, sb, (0,) * NPASS)

    nsrc = scan_cloud(0, 0)
    ntgt = scan_cloud(QSTRIDE, NPASS * WLCAP)

    def splat_atoms(grid, wbase, nhit, s0):
        def body(a, _):
            zc = wlz[pl.ds(wbase + a, L)][0]
            yc = wly[pl.ds(wbase + a, L)][0]
            xc = wlx[pl.ds(wbase + a, L)][0]
            byx = yc * DHW + xc
            myx = []
            for g in range(2):
                y = yc + oy_v[g]
                x = xc + ox_v[g]
                myx.append((y >= 0) & (y < DHW) & (x >= 0) & (x < DHW))
            for lz in range(NZ):
                dz = s0 + lz - zc
                adz = jnp.abs(dz)

                @pl.when(adz <= 2)
                def _(lz=lz, adz=adz):
                    base = lz * PLANE + byx
                    w0 = ftab[pl.ds(adz * 2 * L, L)]
                    idx0 = base + off_v[0]
                    cur0 = plsc.load_gather(grid, [idx0], mask=myx[0])
                    plsc.store_scatter(
                        grid, [idx0], jnp.maximum(cur0, w0), mask=myx[0])

                    @pl.when(adz <= 1)
                    def _(base=base, adz=adz):
                        w1 = ftab[pl.ds((adz * 2 + 1) * L, L)]
                        idx1 = base + off_v[1]
                        cur1 = plsc.load_gather(grid, [idx1], mask=myx[1])
                        plsc.store_scatter(
                            grid, [idx1], jnp.maximum(cur1, w1), mask=myx[1])

            return 0

        lax.fori_loop(0, nhit, body, 0)

    acc_ss = zero_v
    acc_st = zero_v
    for p in range(NPASS):

        def zbody(i, _):
            sg[pl.ds(i * L, L)] = zero_v
            tg[pl.ds(i * L, L)] = zero_v
            return 0

        lax.fori_loop(0, SLAB // L, zbody, 0, unroll=8)

        splat_atoms(sg, p * WLCAP, nsrc[p], s0s[p])
        splat_atoms(tg, (NPASS + p) * WLCAP, ntgt[p], s0s[p])

        def rbody(i, carry):
            css, cst = carry
            s = sg[pl.ds(i * L, L)]
            t = tg[pl.ds(i * L, L)]
            return (css + s * s, cst + s * t)

        acc_ss, acc_st = lax.fori_loop(
            0, SLAB // L, rbody, (acc_ss, acc_st), unroll=8)

    for i in range(2 * DHW // L):
        outv[pl.ds(i * L, L)] = zero_v
    outv[pl.ds(0, L)] = acc_ss
    outv[pl.ds(DHW, L)] = acc_st
    pltpu.sync_copy(outv.at[pl.ds(0, DHW)], out_hbm.at[pl.ds(wid * DHW, DHW)])
    pltpu.sync_copy(outv.at[pl.ds(DHW, DHW)],
                    out_hbm.at[pl.ds((NW + wid) * DHW, DHW)])


@jax.jit
def _run(srcc, tgtt):
    mesh = plsc.VectorSubcoreMesh(
        core_axis_name="c", subcore_axis_name="s", num_cores=NC, num_subcores=NS)
    out = pl.kernel(
        _sc_body,
        out_type=jax.ShapeDtypeStruct((2 * NW * DHW,), jnp.float32),
        mesh=mesh,
        compiler_params=pltpu.CompilerParams(needs_layout_passes=False),
        scratch_types=[
            pltpu.VMEM((SLAB,), jnp.float32),
            pltpu.VMEM((SLAB,), jnp.float32),
            pltpu.VMEM((3 * N_ATOMS,), jnp.float32),
            pltpu.VMEM((2 * QSTRIDE,), jnp.int32),
            pltpu.VMEM((2 * NPASS * WLCAP,), jnp.int32),
            pltpu.VMEM((2 * NPASS * WLCAP,), jnp.int32),
            pltpu.VMEM((2 * NPASS * WLCAP,), jnp.int32),
            pltpu.VMEM((6 * L,), jnp.int32),
            pltpu.VMEM((7 * L,), jnp.float32),
            pltpu.VMEM((2 * DHW,), jnp.float32),
        ],
    )(srcc, tgtt, jnp.asarray(_ITAB), jnp.asarray(_FTAB))
    halves = out.reshape(2, NW * DHW)
    return jnp.sum(halves[0]) - jnp.sum(halves[1])


def kernel(src, tgt):
    return _run(src.reshape(3 * N_ATOMS), tgt.reshape(3 * N_ATOMS))


# speedup vs baseline: 1.7223x; 1.0677x over previous
import numpy as np
import jax
import jax.numpy as jnp
from jax import lax
from jax.experimental import pallas as pl
from jax.experimental.pallas import tpu as pltpu
from jax.experimental.pallas import tpu_sc as plsc

DHW = 128
KCONST = (np.pi / 3.5) ** 2
N_ATOMS = 2000
NC, NS, L = 2, 16, 16
NW = NC * NS
NZ = 2
NPASS = DHW // (NW * NZ)
PLANE = DHW * DHW
SLAB = NZ * PLANE
QSTRIDE = 6144
WLCAP = N_ATOMS + L

_offs = sorted(
    [(oy, ox) for oy in range(-2, 3) for ox in range(-2, 3) if oy * oy + ox * ox <= 6],
    key=lambda p: p[0] * p[0] + p[1] * p[1],
)
_oy = np.array([o[0] for o in _offs] + [0] * 11, np.int32)
_ox = np.array([o[1] for o in _offs] + [0] * 11, np.int32)
_r2 = np.array([o[0] ** 2 + o[1] ** 2 for o in _offs] + [999] * 11, np.int64)

_ITAB = np.concatenate([
    (_oy[0:16] * DHW + _ox[0:16]).astype(np.int32),
    (_oy[16:32] * DHW + _ox[16:32]).astype(np.int32),
    _oy[0:16], _oy[16:32], _ox[0:16], _ox[16:32],
]).astype(np.int32)

_wrow = lambda adz, g: np.where(
    adz * adz + _r2[g * 16:(g + 1) * 16] <= 6,
    np.exp(-KCONST * (adz * adz + _r2[g * 16:(g + 1) * 16].astype(np.float64))),
    0.0).astype(np.float32)
_FTAB = np.concatenate(
    [_wrow(a, g) for a in range(3) for g in range(2)] + [np.zeros(16, np.float32)])


def _sc_body(src_hbm, tgt_hbm, itab_hbm, ftab_hbm, out_hbm,
             sg0, sg1, tg0, tg1, cbuf, wlz, wly, wlx, itab, ftab, outv):
    cid = lax.axis_index("c")
    sid = lax.axis_index("s")
    wid = sid * NC + cid

    pltpu.sync_copy(itab_hbm, itab)
    pltpu.sync_copy(ftab_hbm, ftab)

    off_v = [itab[pl.ds(0, L)], itab[pl.ds(L, L)]]
    oy_v = [itab[pl.ds(2 * L, L)], itab[pl.ds(3 * L, L)]]
    ox_v = [itab[pl.ds(4 * L, L)], itab[pl.ds(5 * L, L)]]
    zero_v = ftab[pl.ds(6 * L, L)]

    s0s = [p * (NW * NZ) + wid * NZ for p in range(NPASS)]

    def scan_cloud(hbm, wbase):
        pltpu.sync_copy(hbm, cbuf)

        def sb(i, cnts):
            zv = (cbuf[pl.ds(i * L, L)] * np.float32(DHW)).astype(jnp.int32)
            yv = (cbuf[pl.ds(N_ATOMS + i * L, L)]
                  * np.float32(DHW)).astype(jnp.int32)
            xv = (cbuf[pl.ds(2 * N_ATOMS + i * L, L)]
                  * np.float32(DHW)).astype(jnp.int32)
            new = []
            for p in range(NPASS):
                h = (zv >= s0s[p] - 2) & (zv <= s0s[p] + NZ + 1)
                c = cnts[p]
                at = wbase + p * WLCAP + c
                plsc.store_compressed(wlz.at[pl.ds(at, L)], zv, mask=h)
                plsc.store_compressed(wly.at[pl.ds(at, L)], yv, mask=h)
                plsc.store_compressed(wlx.at[pl.ds(at, L)], xv, mask=h)
                new.append(c + plsc.all_reduce_population_count(h)[0])
            return tuple(new)

        return lax.fori_loop(0, N_ATOMS // L, sb, (0,) * NPASS)

    nsrc = scan_cloud(src_hbm, 0)
    ntgt = scan_cloud(tgt_hbm, NPASS * WLCAP)

    def splat_atoms(g0, g1, wbase, nhit, s0):
        def body(a, _):
            zc = wlz[pl.ds(wbase + a, L)][0]
            yc = wly[pl.ds(wbase + a, L)][0]
            xc = wlx[pl.ds(wbase + a, L)][0]
            byx = yc * DHW + xc
            myx = []
            for g in range(2):
                y = yc + oy_v[g]
                x = xc + ox_v[g]
                myx.append((y >= 0) & (y < DHW) & (x >= 0) & (x < DHW))
            for lz, grid in ((0, g0), (1, g1)):
                dz = s0 + lz - zc
                adz = jnp.abs(dz)

                @pl.when(adz <= 2)
                def _(grid=grid, adz=adz):
                    w0 = ftab[pl.ds(adz * 2 * L, L)]
                    idx0 = byx + off_v[0]
                    cur0 = plsc.load_gather(grid, [idx0], mask=myx[0])
                    plsc.store_scatter(
                        grid, [idx0], jnp.maximum(cur0, w0), mask=myx[0])

                    @pl.when(adz <= 1)
                    def _(grid=grid, adz=adz):
                        w1 = ftab[pl.ds((adz * 2 + 1) * L, L)]
                        idx1 = byx + off_v[1]
                        cur1 = plsc.load_gather(grid, [idx1], mask=myx[1])
                        plsc.store_scatter(
                            grid, [idx1], jnp.maximum(cur1, w1), mask=myx[1])

            return 0

        lax.fori_loop(0, nhit, body, 0)

    acc_ss = zero_v
    acc_st = zero_v
    for p in range(NPASS):

        def zbody(i, _):
            sg0[pl.ds(i * L, L)] = zero_v
            sg1[pl.ds(i * L, L)] = zero_v
            tg0[pl.ds(i * L, L)] = zero_v
            tg1[pl.ds(i * L, L)] = zero_v
            return 0

        lax.fori_loop(0, PLANE // L, zbody, 0, unroll=8)

        splat_atoms(sg0, sg1, p * WLCAP, nsrc[p], s0s[p])
        splat_atoms(tg0, tg1, (NPASS + p) * WLCAP, ntgt[p], s0s[p])

        def rbody(i, carry):
            css, cst = carry
            s0v = sg0[pl.ds(i * L, L)]
            s1v = sg1[pl.ds(i * L, L)]
            t0v = tg0[pl.ds(i * L, L)]
            t1v = tg1[pl.ds(i * L, L)]
            return (css + s0v * s0v + s1v * s1v,
                    cst + s0v * t0v + s1v * t1v)

        acc_ss, acc_st = lax.fori_loop(
            0, PLANE // L, rbody, (acc_ss, acc_st), unroll=8)

    for i in range(2 * DHW // L):
        outv[pl.ds(i * L, L)] = zero_v
    outv[pl.ds(0, L)] = acc_ss
    outv[pl.ds(DHW, L)] = acc_st
    pltpu.sync_copy(outv.at[pl.ds(0, DHW)], out_hbm.at[pl.ds(wid * DHW, DHW)])
    pltpu.sync_copy(outv.at[pl.ds(DHW, DHW)],
                    out_hbm.at[pl.ds((NW + wid) * DHW, DHW)])


@jax.jit
def _run(srcc, tgtt):
    mesh = plsc.VectorSubcoreMesh(
        core_axis_name="c", subcore_axis_name="s", num_cores=NC, num_subcores=NS)
    out = pl.kernel(
        _sc_body,
        out_type=jax.ShapeDtypeStruct((2 * NW * DHW,), jnp.float32),
        mesh=mesh,
        compiler_params=pltpu.CompilerParams(needs_layout_passes=False),
        scratch_types=[
            pltpu.VMEM((PLANE,), jnp.float32),
            pltpu.VMEM((PLANE,), jnp.float32),
            pltpu.VMEM((PLANE,), jnp.float32),
            pltpu.VMEM((PLANE,), jnp.float32),
            pltpu.VMEM((3 * N_ATOMS,), jnp.float32),
            pltpu.VMEM((2 * NPASS * WLCAP,), jnp.int32),
            pltpu.VMEM((2 * NPASS * WLCAP,), jnp.int32),
            pltpu.VMEM((2 * NPASS * WLCAP,), jnp.int32),
            pltpu.VMEM((6 * L,), jnp.int32),
            pltpu.VMEM((7 * L,), jnp.float32),
            pltpu.VMEM((2 * DHW,), jnp.float32),
        ],
    )(srcc, tgtt, jnp.asarray(_ITAB), jnp.asarray(_FTAB))
    halves = out.reshape(2, NW * DHW)
    return jnp.sum(halves[0]) - jnp.sum(halves[1])


def kernel(src, tgt):
    return _run(src.reshape(3 * N_ATOMS), tgt.reshape(3 * N_ATOMS))
